# baseline (device time: 10404 ns/iter reference)
import jax
import jax.numpy as jnp
from jax import lax
from jax.experimental import pallas as pl
from jax.experimental.pallas import tpu as pltpu

N_DEV = 4


def kernel(x, w_mat):
    m_per, k = x.shape
    _, n = w_mat.shape
    n_per = n // N_DEV
    n_half = n_per // 2

    def body(x_hbm, w_hbm, out_ref, x_vmem, w_buf, chunk_ref, recv_ref,
             load_sems, send_sems, recv_sems):
        my = lax.axis_index("i")

        barrier_sem = pltpu.get_barrier_semaphore()
        for step in range(1, N_DEV):
            pl.semaphore_signal(
                barrier_sem, inc=1,
                device_id=((my + step) % N_DEV,),
                device_id_type=pl.DeviceIdType.MESH,
            )

        order = [2, 1, 3, 0]

        x_dma = pltpu.make_async_copy(x_hbm, x_vmem, load_sems.at[2])
        x_dma.start()

        w_dmas = [None] * N_DEV

        def start_w(kk):
            j = (my + order[kk]) % N_DEV
            d = pltpu.make_async_copy(
                w_hbm.at[:, pl.ds(j * n_per, n_per)],
                w_buf.at[kk % 2],
                load_sems.at[kk % 2],
            )
            d.start()
            w_dmas[kk] = d

        start_w(0)
        x_dma.wait()
        x_val = x_vmem[:, :]

        rdmas = []
        for kk, step in enumerate(order):
            if kk + 1 < N_DEV:
                start_w(kk + 1)
            w_dmas[kk].wait()
            if step == 0:
                y_blk = jnp.maximum(
                    jnp.dot(x_val, w_buf[kk % 2],
                            preferred_element_type=jnp.float32),
                    0.0,
                )
                out_ref[pl.ds(my * m_per, m_per), :] = y_blk
            else:
                j = (my + step) % N_DEV
                for h in range(2):
                    y_half = jnp.maximum(
                        jnp.dot(x_val,
                                w_buf[kk % 2, :, h * n_half:(h + 1) * n_half],
                                preferred_element_type=jnp.float32),
                        0.0,
                    )
                    chunk_ref[kk, h, :, :] = y_half.astype(jnp.bfloat16)
                    if kk == 0 and h == 0:
                        pl.semaphore_wait(barrier_sem, N_DEV - 1)
                    rdma = pltpu.make_async_remote_copy(
                        src_ref=chunk_ref.at[kk, h],
                        dst_ref=recv_ref.at[my, h],
                        send_sem=send_sems.at[kk, h],
                        recv_sem=recv_sems.at[my, h],
                        device_id=(j,),
                        device_id_type=pl.DeviceIdType.MESH,
                    )
                    rdma.start()
                    rdmas.append(rdma)

        for step in [2, 3, 1]:
            j = (my + step) % N_DEV
            for h in range(2):
                recv = pltpu.make_async_remote_copy(
                    src_ref=chunk_ref.at[0, 0],
                    dst_ref=recv_ref.at[j, h],
                    send_sem=send_sems.at[0, 0],
                    recv_sem=recv_sems.at[j, h],
                    device_id=(j,),
                    device_id_type=pl.DeviceIdType.MESH,
                )
                recv.wait_recv()
                out_ref[pl.ds(j * m_per, m_per),
                        h * n_half:(h + 1) * n_half] = (
                    recv_ref[j, h].astype(jnp.float32))

        for rdma in rdmas:
            rdma.wait_send()

    return pl.pallas_call(
        body,
        out_shape=jax.ShapeDtypeStruct((N_DEV * m_per, n_per), jnp.float32),
        in_specs=[
            pl.BlockSpec(memory_space=pltpu.MemorySpace.HBM),
            pl.BlockSpec(memory_space=pltpu.MemorySpace.HBM),
        ],
        out_specs=pl.BlockSpec(memory_space=pltpu.VMEM),
        scratch_shapes=[
            pltpu.VMEM((m_per, k), jnp.float32),
            pltpu.VMEM((2, k, n_per), jnp.float32),
            pltpu.VMEM((N_DEV - 1, 2, m_per, n_half), jnp.bfloat16),
            pltpu.VMEM((N_DEV, 2, m_per, n_half), jnp.bfloat16),
            pltpu.SemaphoreType.DMA((3,)),
            pltpu.SemaphoreType.DMA((N_DEV - 1, 2)),
            pltpu.SemaphoreType.DMA((N_DEV, 2)),
        ],
        compiler_params=pltpu.CompilerParams(collective_id=0),
    )(
        pltpu.with_memory_space_constraint(x, pltpu.MemorySpace.HBM),
        pltpu.with_memory_space_constraint(w_mat, pltpu.MemorySpace.HBM),
    )


# device time: 10314 ns/iter; 1.0087x vs baseline; 1.0087x over previous
import jax
import jax.numpy as jnp
from jax import lax
from jax.experimental import pallas as pl
from jax.experimental.pallas import tpu as pltpu

N_DEV = 4


def kernel(x, w_mat):
    m_per, k = x.shape
    _, n = w_mat.shape
    n_per = n // N_DEV
    n_half = n_per // 2

    def body(x_hbm, w_hbm, out_ref, x_vmem, w_buf, chunk_ref, recv_ref,
             load_sems, send_sems, recv_sems):
        my = lax.axis_index("i")

        barrier_sem = pltpu.get_barrier_semaphore()
        for step in range(1, N_DEV):
            pl.semaphore_signal(
                barrier_sem, inc=1,
                device_id=((my + step) % N_DEV,),
                device_id_type=pl.DeviceIdType.MESH,
            )

        order = [2, 1, 3, 0]

        x_dma = pltpu.make_async_copy(x_hbm, x_vmem, load_sems.at[2])
        x_dma.start()

        w_dmas = [None] * N_DEV

        def start_w(kk):
            j = (my + order[kk]) % N_DEV
            d = pltpu.make_async_copy(
                w_hbm.at[:, pl.ds(j * n_per, n_per)],
                w_buf.at[kk % 2],
                load_sems.at[kk % 2],
            )
            d.start()
            w_dmas[kk] = d

        j0 = (my + order[0]) % N_DEV
        w0h_dmas = []
        for h in range(2):
            d = pltpu.make_async_copy(
                w_hbm.at[:, pl.ds(j0 * n_per + h * n_half, n_half)],
                w_buf.at[0, :, pl.ds(h * n_half, n_half)],
                load_sems.at[0 if h == 0 else 3],
            )
            d.start()
            w0h_dmas.append(d)
        x_dma.wait()
        x_val = x_vmem[:, :]

        rdmas = []
        for kk, step in enumerate(order):
            if kk + 1 < N_DEV:
                start_w(kk + 1)
            if kk == 0:
                w0h_dmas[0].wait()
            else:
                w_dmas[kk].wait()
            if step == 0:
                y_blk = jnp.maximum(
                    jnp.dot(x_val, w_buf[kk % 2],
                            preferred_element_type=jnp.float32),
                    0.0,
                )
                out_ref[pl.ds(my * m_per, m_per), :] = y_blk
            else:
                j = (my + step) % N_DEV
                for h in range(2):
                    y_half = jnp.maximum(
                        jnp.dot(x_val,
                                w_buf[kk % 2, :, h * n_half:(h + 1) * n_half],
                                preferred_element_type=jnp.float32),
                        0.0,
                    )
                    chunk_ref[kk, h, :, :] = y_half.astype(jnp.bfloat16)
                    if kk == 0 and h == 0:
                        pl.semaphore_wait(barrier_sem, N_DEV - 1)
                        w0h_dmas[1].wait()
                    rdma = pltpu.make_async_remote_copy(
                        src_ref=chunk_ref.at[kk, h],
                        dst_ref=recv_ref.at[my, h],
                        send_sem=send_sems.at[kk, h],
                        recv_sem=recv_sems.at[my, h],
                        device_id=(j,),
                        device_id_type=pl.DeviceIdType.MESH,
                    )
                    rdma.start()
                    rdmas.append(rdma)

        for step in [2, 3, 1]:
            j = (my + step) % N_DEV
            for h in range(2):
                recv = pltpu.make_async_remote_copy(
                    src_ref=chunk_ref.at[0, 0],
                    dst_ref=recv_ref.at[j, h],
                    send_sem=send_sems.at[0, 0],
                    recv_sem=recv_sems.at[j, h],
                    device_id=(j,),
                    device_id_type=pl.DeviceIdType.MESH,
                )
                recv.wait_recv()
                out_ref[pl.ds(j * m_per, m_per),
                        h * n_half:(h + 1) * n_half] = (
                    recv_ref[j, h].astype(jnp.float32))

        for rdma in rdmas:
            rdma.wait_send()

    return pl.pallas_call(
        body,
        out_shape=jax.ShapeDtypeStruct((N_DEV * m_per, n_per), jnp.float32),
        in_specs=[
            pl.BlockSpec(memory_space=pltpu.MemorySpace.HBM),
            pl.BlockSpec(memory_space=pltpu.MemorySpace.HBM),
        ],
        out_specs=pl.BlockSpec(memory_space=pltpu.VMEM),
        scratch_shapes=[
            pltpu.VMEM((m_per, k), jnp.float32),
            pltpu.VMEM((2, k, n_per), jnp.float32),
            pltpu.VMEM((N_DEV - 1, 2, m_per, n_half), jnp.bfloat16),
            pltpu.VMEM((N_DEV, 2, m_per, n_half), jnp.bfloat16),
            pltpu.SemaphoreType.DMA((4,)),
            pltpu.SemaphoreType.DMA((N_DEV - 1, 2)),
            pltpu.SemaphoreType.DMA((N_DEV, 2)),
        ],
        compiler_params=pltpu.CompilerParams(collective_id=0),
    )(
        pltpu.with_memory_space_constraint(x, pltpu.MemorySpace.HBM),
        pltpu.with_memory_space_constraint(w_mat, pltpu.MemorySpace.HBM),
    )


# device time: 10311 ns/iter; 1.0090x vs baseline; 1.0003x over previous
import jax
import jax.numpy as jnp
from jax import lax
from jax.experimental import pallas as pl
from jax.experimental.pallas import tpu as pltpu

N_DEV = 4


def kernel(x, w_mat):
    m_per, k = x.shape
    _, n = w_mat.shape
    n_per = n // N_DEV
    n_half = n_per // 2

    def body(x_hbm, w_hbm, out_ref, x_vmem, w_buf, chunk_ref, recv_ref,
             load_sems, send_sems, recv_sems):
        my = lax.axis_index("i")

        barrier_sem = pltpu.get_barrier_semaphore()
        for step in range(1, N_DEV):
            pl.semaphore_signal(
                barrier_sem, inc=1,
                device_id=((my + step) % N_DEV,),
                device_id_type=pl.DeviceIdType.MESH,
            )

        order = [2, 1, 3, 0]

        x_dma = pltpu.make_async_copy(x_hbm, x_vmem, load_sems.at[2])
        x_dma.start()

        w_dmas = [None] * N_DEV

        def start_w(kk):
            j = (my + order[kk]) % N_DEV
            d = pltpu.make_async_copy(
                w_hbm.at[:, pl.ds(j * n_per, n_per)],
                w_buf.at[kk % 2],
                load_sems.at[kk % 2],
            )
            d.start()
            w_dmas[kk] = d

        j0 = (my + order[0]) % N_DEV
        w0h_dmas = []
        for h in range(2):
            d = pltpu.make_async_copy(
                w_hbm.at[:, pl.ds(j0 * n_per + h * n_half, n_half)],
                w_buf.at[0, :, pl.ds(h * n_half, n_half)],
                load_sems.at[0 if h == 0 else 3],
            )
            d.start()
            w0h_dmas.append(d)
        x_dma.wait()
        x_val = x_vmem[:, :]

        rdmas = []
        for kk, step in enumerate(order):
            if kk + 1 < N_DEV:
                start_w(kk + 1)
            if kk == 0:
                w0h_dmas[0].wait()
            else:
                w_dmas[kk].wait()
            if step == 0:
                y_blk = jnp.maximum(
                    jnp.dot(x_val, w_buf[kk % 2],
                            preferred_element_type=jnp.float32),
                    0.0,
                )
                out_ref[pl.ds(my * m_per, m_per), :] = y_blk
            else:
                j = (my + step) % N_DEV
                for h in range(2):
                    y_half = jnp.maximum(
                        jnp.dot(x_val,
                                w_buf[kk % 2, :, h * n_half:(h + 1) * n_half],
                                preferred_element_type=jnp.float32),
                        0.0,
                    )
                    chunk_ref[kk, h, :, :] = y_half.astype(jnp.bfloat16)
                    if kk == 0 and h == 0:
                        pl.semaphore_wait(barrier_sem, N_DEV - 1)
                        w0h_dmas[1].wait()
                    rdma = pltpu.make_async_remote_copy(
                        src_ref=chunk_ref.at[kk, h],
                        dst_ref=recv_ref.at[my, h],
                        send_sem=send_sems.at[kk, h],
                        recv_sem=recv_sems.at[my, h],
                        device_id=(j,),
                        device_id_type=pl.DeviceIdType.MESH,
                    )
                    rdma.start()
                    rdmas.append(rdma)

        for step in [3, 2, 1]:
            j = (my + step) % N_DEV
            for h in range(2):
                recv = pltpu.make_async_remote_copy(
                    src_ref=chunk_ref.at[0, 0],
                    dst_ref=recv_ref.at[j, h],
                    send_sem=send_sems.at[0, 0],
                    recv_sem=recv_sems.at[j, h],
                    device_id=(j,),
                    device_id_type=pl.DeviceIdType.MESH,
                )
                recv.wait_recv()
                out_ref[pl.ds(j * m_per, m_per),
                        h * n_half:(h + 1) * n_half] = (
                    recv_ref[j, h].astype(jnp.float32))

        for rdma in rdmas:
            rdma.wait_send()

    return pl.pallas_call(
        body,
        out_shape=jax.ShapeDtypeStruct((N_DEV * m_per, n_per), jnp.float32),
        in_specs=[
            pl.BlockSpec(memory_space=pltpu.MemorySpace.HBM),
            pl.BlockSpec(memory_space=pltpu.MemorySpace.HBM),
        ],
        out_specs=pl.BlockSpec(memory_space=pltpu.VMEM),
        scratch_shapes=[
            pltpu.VMEM((m_per, k), jnp.float32),
            pltpu.VMEM((2, k, n_per), jnp.float32),
            pltpu.VMEM((N_DEV - 1, 2, m_per, n_half), jnp.bfloat16),
            pltpu.VMEM((N_DEV, 2, m_per, n_half), jnp.bfloat16),
            pltpu.SemaphoreType.DMA((4,)),
            pltpu.SemaphoreType.DMA((N_DEV - 1, 2)),
            pltpu.SemaphoreType.DMA((N_DEV, 2)),
        ],
        compiler_params=pltpu.CompilerParams(collective_id=0),
    )(
        pltpu.with_memory_space_constraint(x, pltpu.MemorySpace.HBM),
        pltpu.with_memory_space_constraint(w_mat, pltpu.MemorySpace.HBM),
    )
